# baseline (speedup 1.0000x reference)
import functools

import jax
import jax.numpy as jnp
from jax import lax
from jax.experimental import pallas as pl
from jax.experimental.pallas import tpu as pltpu
from jax.experimental.pallas import tpu_sc as plsc

INPUT_DIM = 1000000
EMBED_DIM = 32
B = 16384
T = 50
ROWS_PER_LINE = 128 // EMBED_DIM
NLINES = INPUT_DIM // ROWS_PER_LINE

NC, NS = 2, 16
NW = NC * NS
BT = 128
ET = 8
K = 4
B_PER_W = K * BT
NBT = B // BT


def _make_kernel():
    mesh = plsc.VectorSubcoreMesh(core_axis_name="c", subcore_axis_name="s")

    @functools.partial(
        pl.kernel,
        out_type=jax.ShapeDtypeStruct((T, EMBED_DIM // ET, NBT, ET, BT),
                                      jnp.float32),
        mesh=mesh,
        scratch_types=[
            pltpu.VMEM((B_PER_W // 2, T), jnp.int32),
            pltpu.VMEM((T, K, BT), jnp.int32),
            pltpu.VMEM((T, K, BT), jnp.int32),
            pltpu.VMEM((2, BT // 2, 128), jnp.float32),
            pltpu.VMEM((2, EMBED_DIM, BT + 1), jnp.float32),
            pltpu.SemaphoreType.DMA,
            pltpu.SemaphoreType.DMA,
        ],
        compiler_params=pltpu.CompilerParams(use_tc_tiling_on_sc=True,
                                             needs_layout_passes=False),
    )
    def emb(x_hbm, table_hbm, out_hbm, idx_v, idxT_v, offT_v, gbufs, sbufs,
            gsem, wsem):
        wid = lax.axis_index("s") * NC + lax.axis_index("c")
        base_b = wid * B_PER_W
        iota = jax.lax.iota(jnp.int32, 16)
        erows = [iota + h * 16 for h in range(2)]
        HB = BT // 2
        NHB = 2 * K

        for half in range(2):
            pltpu.sync_copy(
                x_hbm.at[pl.ds(base_b + half * (B_PER_W // 2), B_PER_W // 2)],
                idx_v)

            @pl.loop(0, T)
            def _(t):
                tcol = jnp.full((16,), t, jnp.int32)
                for kk in range(K // 2):
                    k = half * 2 + kk
                    for g in range(BT // 16):
                        rows = iota + (kk * BT + g * 16)
                        v = plsc.load_gather(idx_v, [rows, tcol])
                        idxT_v[t, k, pl.ds(g * 16, 16)] = v >> 2
                        offT_v[t, k, pl.ds(g * 16, 16)] = (v & 3) * EMBED_DIM

        def gstart(t, i, slot):
            pltpu.make_async_copy(
                table_hbm.at[idxT_v.at[t, i // 2, pl.ds((i % 2) * HB, HB)]],
                gbufs.at[slot], gsem).start()

        def gwait(slot):
            pltpu.make_async_copy(
                table_hbm.at[pl.ds(0, HB)], gbufs.at[slot], gsem).wait()

        def wdesc(t, k, s, et):
            return pltpu.make_async_copy(
                sbufs.at[s, pl.ds(et * ET, ET), pl.ds(0, BT)],
                out_hbm.at[t, et, wid * K + k], wsem)

        gstart(0, 0, 0)
        gstart(0, 1, 1)

        @pl.loop(0, T)
        def _(t):
            for i in range(NHB):
                k, h, slot = i // 2, i % 2, i % 2
                s = k % 2
                gwait(slot)
                if h == 0:
                    if k < 2:
                        @pl.when(t > 0)
                        def _():
                            for et in range(EMBED_DIM // ET):
                                wdesc(t - 1, k + 2, s, et).wait()
                    else:
                        for et in range(EMBED_DIM // ET):
                            wdesc(t, k - 2, s, et).wait()

                @pl.loop(0, HB // 16)
                def _(g):
                    offv = offT_v[t, k, pl.ds(h * HB + g * 16, 16)]
                    for bi in range(16):
                        b = g * 16 + bi
                        bb = jnp.full((16,), h * HB + b, jnp.int32)
                        off = offv[bi]
                        for hh in range(2):
                            v = gbufs[slot, b, pl.ds(off + hh * 16, 16)]
                            plsc.store_scatter(
                                sbufs.at[s], [erows[hh], bb], v)

                if h == 1:
                    for et in range(EMBED_DIM // ET):
                        wdesc(t, k, s, et).start()

                if i < NHB - 2:
                    gstart(t, i + 2, slot)
                else:
                    @pl.when(t < T - 1)
                    def _():
                        gstart(t + 1, i - (NHB - 2), slot)

        for k in (2, 3):
            for et in range(EMBED_DIM // ET):
                wdesc(T - 1, k, k % 2, et).wait()

    return emb


_emb_kernel = _make_kernel()


def kernel(x, wts):
    lines = wts.reshape(NLINES, 128)
    out5 = _emb_kernel(x, lines)
    return out5.transpose(2, 4, 0, 1, 3).reshape(B, T, EMBED_DIM)


# speedup vs baseline: 1.4306x; 1.4306x over previous
import functools

import jax
import jax.numpy as jnp
from jax import lax
from jax.experimental import pallas as pl
from jax.experimental.pallas import tpu as pltpu
from jax.experimental.pallas import tpu_sc as plsc

INPUT_DIM = 1000000
EMBED_DIM = 32
B = 16384
T = 50

NC, NS = 2, 16
NW = NC * NS
BT = 128
ET = 8
K = 4
B_PER_W = K * BT
NBT = B // BT


def _make_kernel():
    mesh = plsc.VectorSubcoreMesh(core_axis_name="c", subcore_axis_name="s")

    @functools.partial(
        pl.kernel,
        out_type=jax.ShapeDtypeStruct((T, EMBED_DIM // ET, NBT, ET, BT),
                                      jnp.float32),
        mesh=mesh,
        scratch_types=[
            pltpu.VMEM((B_PER_W, T), jnp.int32),
            pltpu.VMEM((T, K, BT), jnp.int32),
            pltpu.VMEM((K, BT, EMBED_DIM), jnp.float32),
            pltpu.VMEM((2, EMBED_DIM // ET, ET, BT + 1), jnp.float32),
            pltpu.SemaphoreType.DMA,
            pltpu.SemaphoreType.DMA,
        ],
        compiler_params=pltpu.CompilerParams(use_tc_tiling_on_sc=False,
                                             needs_layout_passes=False),
    )
    def emb(x_hbm, table_hbm, out_hbm, idx_v, idxT_v, gbufs, sbufs,
            gsem, wsem):
        wid = lax.axis_index("s") * NC + lax.axis_index("c")
        base_b = wid * B_PER_W
        iota = jax.lax.iota(jnp.int32, 16)
        erows_hi = [(iota + h * 16) >> 3 for h in range(2)]
        erows_lo = [(iota + h * 16) & 7 for h in range(2)]

        pltpu.sync_copy(x_hbm.at[pl.ds(base_b, B_PER_W)], idx_v)

        @pl.loop(0, T)
        def _(t):
            tcol = jnp.full((16,), t, jnp.int32)
            for k in range(K):
                for g in range(BT // 16):
                    rows = iota + (k * BT + g * 16)
                    v = plsc.load_gather(idx_v, [rows, tcol])
                    idxT_v[t, k, pl.ds(g * 16, 16)] = v

        def gstart(t, k):
            pltpu.make_async_copy(
                table_hbm.at[idxT_v.at[t, k]], gbufs.at[k], gsem).start()

        def gwait(k):
            pltpu.make_async_copy(
                table_hbm.at[pl.ds(0, BT)], gbufs.at[k], gsem).wait()

        def wdesc(t, k, s):
            return pltpu.make_async_copy(
                sbufs.at[s, :, :, pl.ds(0, BT)],
                out_hbm.at[t, :, wid * K + k], wsem)

        for k in range(K):
            gstart(0, k)

        @pl.loop(0, T)
        def _(t):
            for k in range(K):
                s = k % 2
                gwait(k)
                if k < 2:
                    @pl.when(t > 0)
                    def _():
                        wdesc(t - 1, k + 2, s).wait()
                else:
                    wdesc(t, k - 2, s).wait()

                @pl.loop(0, BT, unroll=16)
                def _(b):
                    bb = jnp.full((16,), b, jnp.int32)
                    for h in range(2):
                        v = gbufs[k, b, pl.ds(h * 16, 16)]
                        plsc.store_scatter(
                            sbufs.at[s], [erows_hi[h], erows_lo[h], bb], v)

                wdesc(t, k, s).start()

                @pl.when(t < T - 1)
                def _():
                    gstart(t + 1, k)

        for k in (2, 3):
            wdesc(T - 1, k, k % 2).wait()

    return emb


_emb_kernel = _make_kernel()


def kernel(x, wts):
    out5 = _emb_kernel(x, wts)
    return out5.transpose(2, 4, 0, 1, 3).reshape(B, T, EMBED_DIM)


# speedup vs baseline: 1.5109x; 1.0562x over previous
import functools

import jax
import jax.numpy as jnp
from jax import lax
from jax.experimental import pallas as pl
from jax.experimental.pallas import tpu as pltpu
from jax.experimental.pallas import tpu_sc as plsc

INPUT_DIM = 1000000
EMBED_DIM = 32
B = 16384
T = 50

NC, NS = 2, 16
NW = NC * NS
BT = 128
ET = 8
K = 4
B_PER_W = K * BT
NBT = B // BT


def _make_kernel():
    mesh = plsc.VectorSubcoreMesh(core_axis_name="c", subcore_axis_name="s")

    @functools.partial(
        pl.kernel,
        out_type=jax.ShapeDtypeStruct((T, EMBED_DIM // ET, NBT, ET, BT),
                                      jnp.float32),
        mesh=mesh,
        scratch_types=[
            pltpu.VMEM((B_PER_W, T), jnp.int32),
            pltpu.VMEM((T, K, BT), jnp.int32),
            pltpu.VMEM((K, BT, EMBED_DIM), jnp.float32),
            pltpu.VMEM((2, EMBED_DIM, BT + 1), jnp.float32),
            pltpu.SemaphoreType.DMA,
            pltpu.SemaphoreType.DMA,
        ],
        compiler_params=pltpu.CompilerParams(use_tc_tiling_on_sc=False,
                                             needs_layout_passes=False),
    )
    def emb(x_hbm, table_hbm, out_hbm, idx_v, idxT_v, gbufs, sbufs,
            gsem, wsem):
        wid = lax.axis_index("s") * NC + lax.axis_index("c")
        base_b = wid * B_PER_W
        iota = jax.lax.iota(jnp.int32, 16)
        erows = [iota + h * 16 for h in range(2)]

        pltpu.sync_copy(x_hbm.at[pl.ds(base_b, B_PER_W)], idx_v)

        @pl.loop(0, T)
        def _(t):
            tcol = jnp.full((16,), t, jnp.int32)
            for k in range(K):
                for g in range(BT // 16):
                    rows = iota + (k * BT + g * 16)
                    v = plsc.load_gather(idx_v, [rows, tcol])
                    idxT_v[t, k, pl.ds(g * 16, 16)] = v

        def gstart(t, k):
            pltpu.make_async_copy(
                table_hbm.at[idxT_v.at[t, k]], gbufs.at[k], gsem).start()

        def gwait(k):
            pltpu.make_async_copy(
                table_hbm.at[pl.ds(0, BT)], gbufs.at[k], gsem).wait()

        def wdesc(t, k, s, et):
            return pltpu.make_async_copy(
                sbufs.at[s, pl.ds(et * ET, ET), pl.ds(0, BT)],
                out_hbm.at[t, et, wid * K + k], wsem)

        for k in range(K):
            gstart(0, k)

        @pl.loop(0, T)
        def _(t):
            for k in range(K):
                s = k % 2
                gwait(k)
                if k < 2:
                    @pl.when(t > 0)
                    def _():
                        for et in range(EMBED_DIM // ET):
                            wdesc(t - 1, k + 2, s, et).wait()
                else:
                    for et in range(EMBED_DIM // ET):
                        wdesc(t, k - 2, s, et).wait()

                @pl.loop(0, BT, unroll=8)
                def _(b):
                    bb = jnp.full((16,), b, jnp.int32)
                    for h in range(2):
                        v = gbufs[k, b, pl.ds(h * 16, 16)]
                        plsc.store_scatter(sbufs.at[s], [erows[h], bb], v)

                for et in range(EMBED_DIM // ET):
                    wdesc(t, k, s, et).start()

                @pl.when(t < T - 1)
                def _():
                    gstart(t + 1, k)

        for k in (2, 3):
            for et in range(EMBED_DIM // ET):
                wdesc(T - 1, k, k % 2, et).wait()

    return emb


_emb_kernel = _make_kernel()


def kernel(x, wts):
    out5 = _emb_kernel(x, wts)
    return out5.transpose(2, 4, 0, 1, 3).reshape(B, T, EMBED_DIM)
